# baseline (speedup 1.0000x reference)
import jax
import jax.numpy as jnp
from jax import lax
from jax.experimental import pallas as pl
from jax.experimental.pallas import tpu as pltpu
from jax.experimental.pallas import tpu_sc as plsc

DIM = 128
BATCH = 512
L = 16
NC, NS = 2, 16
NW = NC * NS
RPW = BATCH // NW
NCHUNK = DIM // L
NQ = 9


def _sqrt16(s):
    s = jnp.maximum(s, 1e-35)
    i = plsc.bitcast(s, jnp.int32)
    y = plsc.bitcast(jnp.int32(0x5F3759DF) - (i >> 1), jnp.float32)
    for _ in range(3):
        y = y * (1.5 - 0.5 * s * y * y)
    return s * y


def _sc_body(cE, rE, idx_all, out,
             ib,
             a1, b1, a2, b2, e2b, a3, b3, r3, a4, b4, r4,
             adj, bdj, ang, bng, rng,
             pa, pb, pn, acc_out,
             isem, osem, sems):
    cid = lax.axis_index("c")
    sid = lax.axis_index("s")
    wid = sid * NC + cid
    base = wid * RPW
    iota = lax.iota(jnp.int32, L)
    zero = jnp.zeros((L,), jnp.float32)

    icp = pltpu.make_async_copy(idx_all.at[pl.ds(base, RPW)], ib, isem)
    icp.start()
    icp.wait()

    def col(j):
        return plsc.load_gather(ib, [iota, jnp.full((L,), j, jnp.int32)])

    plans = [
        (sems.at[0], ((cE, col(0), a1), (cE, col(1), b1))),
        (sems.at[1], ((cE, col(11), adj), (cE, col(12), bdj))),
        (sems.at[2], ((cE, col(5), a3), (cE, col(7), b3), (rE, col(6), r3))),
        (sems.at[3], ((cE, col(13), ang), (cE, col(15), bng),
                      (rE, col(14), rng))),
        (sems.at[4], ((cE, col(9), a4), (cE, col(10), b4), (rE, col(8), r4))),
        (sems.at[5], ((cE, col(2), a2), (cE, col(3), b2), (cE, col(4), e2b))),
    ]
    started = []
    for sem, gathers in plans:
        cps = [pltpu.make_async_copy(tab.at[ix], buf, sem)
               for tab, ix, buf in gathers]
        for cp in cps:
            cp.start()
        started.append(cps)

    def wait(k):
        for cp in started[k]:
            cp.wait()

    def cc_total(cbuf, dbuf, rbuf, r_sign, co_sign):
        def row(i, carry):
            def chunk(k, inner):
                accs = []
                for h, acc in enumerate(inner):
                    kk = 2 * k + h
                    c1 = cbuf[i, pl.ds(kk * L, L)]
                    d1 = dbuf[i, pl.ds(kk * L, L)]
                    co = jnp.abs(cbuf[i, pl.ds(DIM + kk * L, L)])
                    do = jnp.abs(dbuf[i, pl.ds(DIM + kk * L, L)])
                    cen = c1 - d1
                    if rbuf is not None:
                        r = rbuf[i, pl.ds(kk * L, L)]
                        cen = cen + r if r_sign > 0 else cen - r
                    euc = jnp.abs(cen)
                    if co_sign > 0:
                        t = jnp.maximum(euc + co - do, 0.0)
                    else:
                        t = jnp.maximum(euc - co - do, 0.0)
                    accs.append(acc + t * t)
                return tuple(accs)
            return lax.fori_loop(0, NCHUNK // 2, chunk, carry, unroll=True)
        acc0, acc1 = lax.fori_loop(0, RPW, row, (zero, zero))
        return acc0 + acc1

    def colsum(pbuf):
        tot = zero
        for c in range(L):
            tot = tot + plsc.load_gather(pbuf, [iota * L + c])
        return tot

    wait(0)
    v1 = cc_total(a1, b1, None, 0, +1)
    wait(1)

    def dj_row(i, carry):
        def chunk(k, inner):
            accs = []
            for h, acc in enumerate(inner):
                kk = 2 * k + h
                c1 = adj[i, pl.ds(kk * L, L)]
                d1 = bdj[i, pl.ds(kk * L, L)]
                co = jnp.abs(adj[i, pl.ds(DIM + kk * L, L)])
                do = jnp.abs(bdj[i, pl.ds(DIM + kk * L, L)])
                t = jnp.maximum(co + do - jnp.abs(c1 - d1), 0.0)
                accs.append(acc + t * t)
            return tuple(accs)
        return lax.fori_loop(0, NCHUNK // 2, chunk, carry, unroll=True)
    dj0, dj1 = lax.fori_loop(0, RPW, dj_row, (zero, zero))
    vdj = dj0 + dj1

    wait(2)
    v3 = cc_total(a3, b3, r3, +1, +1)
    wait(3)

    def ng_row(i, _):
        def chunk(k, inner):
            accs = []
            for h, acc in enumerate(inner):
                kk = 2 * k + h
                c1 = ang[i, pl.ds(kk * L, L)]
                d1 = bng[i, pl.ds(kk * L, L)]
                co = jnp.abs(ang[i, pl.ds(DIM + kk * L, L)])
                do = jnp.abs(bng[i, pl.ds(DIM + kk * L, L)])
                r = rng[i, pl.ds(kk * L, L)]
                t = jnp.maximum(jnp.abs(c1 + r - d1) - co - do, 0.0)
                accs.append(acc + t * t)
            return tuple(accs)
        n0, n1 = lax.fori_loop(0, NCHUNK // 2, chunk, (zero, zero),
                               unroll=True)
        pn[pl.ds(i * L, L)] = n0 + n1
        return 0
    lax.fori_loop(0, RPW, ng_row, 0)

    wait(4)
    v4 = cc_total(a4, b4, r4, -1, -1)
    wait(5)

    def nf2_row(i, _):
        def chunk(k, carry):
            aa, bb = carry
            c1 = a2[i, pl.ds(k * L, L)]
            d1 = b2[i, pl.ds(k * L, L)]
            e1 = e2b[i, pl.ds(k * L, L)]
            c2 = jnp.abs(a2[i, pl.ds(DIM + k * L, L)])
            d2 = jnp.abs(b2[i, pl.ds(DIM + k * L, L)])
            e2 = jnp.abs(e2b[i, pl.ds(DIM + k * L, L)])
            start = jnp.maximum(c1 - c2, d1 - d2)
            end = jnp.minimum(c1 + c2, d1 + d2)
            diff = start - end
            new_r = jnp.abs(diff) * 0.5
            cen1 = (start + end) * 0.5
            u = jnp.maximum(jnp.abs(cen1 - e1) + new_r - e2, 0.0)
            v = jnp.maximum(diff, 0.0)
            return aa + u * u, bb + v * v
        aa, bb = lax.fori_loop(0, NCHUNK, chunk, (zero, zero), unroll=True)
        pa[pl.ds(i * L, L)] = aa
        pb[pl.ds(i * L, L)] = bb
        return 0
    lax.fori_loop(0, RPW, nf2_row, 0)

    a2row = colsum(pa)
    b2row = colsum(pb)
    n2row = colsum(pn)
    va = _sqrt16(a2row)
    vb = _sqrt16(b2row)
    vneg = n2row - 4.0 * _sqrt16(n2row) + 4.0

    acc_out[0, :] = v1
    acc_out[1, :] = vdj
    acc_out[2, :] = v3
    acc_out[3, :] = v4
    acc_out[4, :] = a2row
    acc_out[5, :] = b2row
    acc_out[6, :] = va
    acc_out[7, :] = vb
    acc_out[8, :] = vneg
    ocp = pltpu.make_async_copy(acc_out, out.at[wid], osem)
    ocp.start()
    ocp.wait()


_cbuf = pltpu.VMEM((RPW, 2 * DIM), jnp.float32)
_rbuf = pltpu.VMEM((RPW, DIM), jnp.float32)
_pbuf = pltpu.VMEM((RPW * L,), jnp.float32)


import functools


@functools.cache
def _make_sc_kernel():
    return pl.kernel(
    _sc_body,
    out_type=jax.ShapeDtypeStruct((NW, NQ, L), jnp.float32),
    mesh=plsc.VectorSubcoreMesh(core_axis_name="c", subcore_axis_name="s"),
    compiler_params=pltpu.CompilerParams(needs_layout_passes=False),
    scratch_types=[
        pltpu.VMEM((RPW, 16), jnp.int32),
        _cbuf, _cbuf,
        _cbuf, _cbuf, _cbuf,
        _cbuf, _cbuf, _rbuf,
        _cbuf, _cbuf, _rbuf,
        _cbuf, _cbuf,
        _cbuf, _cbuf, _rbuf,
        _pbuf, _pbuf, _pbuf,
        pltpu.VMEM((NQ, L), jnp.float32),
        pltpu.SemaphoreType.DMA,
        pltpu.SemaphoreType.DMA,
        pltpu.SemaphoreType.DMA((6,)),
    ],
)


def kernel(classEmb, relEmb, nf1, nf2, nf3, nf4, disjoint, nf3_neg):
    idx_all = jnp.concatenate(
        [nf1[:BATCH], nf2[:BATCH], nf3[:BATCH], nf4[:BATCH],
         disjoint[:BATCH], nf3_neg[:BATCH]], axis=1)
    parts = _make_sc_kernel()(classEmb, relEmb, idx_all)
    q = jnp.sum(parts, axis=(0, 2))
    inv_b = 1.0 / BATCH
    loss1 = q[0] * inv_b
    dj = q[1] * inv_b
    loss3 = q[2] * inv_b
    loss4 = q[3] * inv_b
    loss2 = (q[4] + q[5]) * inv_b + 2.0 * (q[6] * inv_b) * (q[7] * inv_b)
    neg = q[8] * inv_b
    return loss1 + loss2 + dj + loss3 + loss4 + neg
